# baseline (speedup 1.0000x reference)
import jax
import jax.numpy as jnp
from jax import lax
from jax.experimental import pallas as pl
from jax.experimental.pallas import tpu as pltpu
from jax.experimental.pallas import tpu_sc as plsc

BATCH = 16384
NUM_ROWS = 1000000
EMBED_DIM = 32
NUM_CORES = 2
NUM_SUBCORES = 16
LANES = 16
NUM_WORKERS = NUM_CORES * NUM_SUBCORES
BPW = BATCH // NUM_WORKERS


def _body(uid_hbm, iid_hbm, uflat_hbm, iflat_hbm, ub_hbm, ib_hbm, out_hbm,
          uidx_v, iidx_v, fuidx_v, fiidx_v, ucols_v, icols_v,
          ub_v, ib_v, out_v, sem, bsem):
    wid = lax.axis_index("s") * NUM_CORES + lax.axis_index("c")
    base = wid * BPW

    pltpu.sync_copy(uid_hbm.at[pl.ds(base, BPW)], uidx_v)
    pltpu.sync_copy(iid_hbm.at[pl.ds(base, BPW)], iidx_v)

    cp_ub = pltpu.async_copy(ub_hbm.at[uidx_v], ub_v, bsem)
    cp_ib = pltpu.async_copy(ib_hbm.at[iidx_v], ib_v, bsem)

    def fill(d, carry):
        off = d * NUM_ROWS

        def chunk(j, c2):
            src = pl.ds(j * LANES, LANES)
            dst = pl.ds(d * BPW + j * LANES, LANES)
            fuidx_v[dst] = uidx_v[src] + off
            fiidx_v[dst] = iidx_v[src] + off
            return c2

        lax.fori_loop(0, BPW // LANES, chunk, 0)
        return carry

    lax.fori_loop(0, EMBED_DIM, fill, 0)

    cp_u = pltpu.async_copy(uflat_hbm.at[fuidx_v], ucols_v, sem)
    cp_i = pltpu.async_copy(iflat_hbm.at[fiidx_v], icols_v, sem)
    cp_u.wait()
    cp_i.wait()
    cp_ub.wait()
    cp_ib.wait()

    def group_body(g, carry):
        s = g * LANES
        acc = ub_v[pl.ds(s, LANES)] + ib_v[pl.ds(s, LANES)]
        for d in range(EMBED_DIM):
            acc = acc + (ucols_v[pl.ds(d * BPW + s, LANES)]
                         * icols_v[pl.ds(d * BPW + s, LANES)])
        out_v[pl.ds(s, LANES)] = acc
        return carry

    lax.fori_loop(0, BPW // LANES, group_body, 0)

    pltpu.sync_copy(out_v, out_hbm.at[pl.ds(base, BPW)])


@jax.jit
def _mf_scores(uid, iid, uflat, iflat, ub, ib):
    mesh = plsc.VectorSubcoreMesh(core_axis_name="c", subcore_axis_name="s")
    return pl.kernel(
        _body,
        out_type=jax.ShapeDtypeStruct((BATCH,), jnp.float32),
        mesh=mesh,
        compiler_params=pltpu.CompilerParams(needs_layout_passes=False),
        scratch_types=[
            pltpu.VMEM((BPW,), jnp.int32),
            pltpu.VMEM((BPW,), jnp.int32),
            pltpu.VMEM((EMBED_DIM * BPW,), jnp.int32),
            pltpu.VMEM((EMBED_DIM * BPW,), jnp.int32),
            pltpu.VMEM((EMBED_DIM * BPW,), jnp.float32),
            pltpu.VMEM((EMBED_DIM * BPW,), jnp.float32),
            pltpu.VMEM((BPW,), jnp.float32),
            pltpu.VMEM((BPW,), jnp.float32),
            pltpu.VMEM((BPW,), jnp.float32),
            pltpu.SemaphoreType.DMA,
            pltpu.SemaphoreType.DMA,
        ],
    )(uid, iid, uflat, iflat, ub, ib)


def kernel(user_ids, item_ids, user_emb, item_emb, user_bias, item_bias):
    uid = user_ids.astype(jnp.int32)
    iid = item_ids.astype(jnp.int32)
    return _mf_scores(uid, iid,
                      user_emb.T.reshape(-1), item_emb.T.reshape(-1),
                      user_bias.reshape(-1), item_bias.reshape(-1))


# speedup vs baseline: 2.3874x; 2.3874x over previous
import jax
import jax.numpy as jnp
from jax import lax
from jax.experimental import pallas as pl
from jax.experimental.pallas import tpu as pltpu
from jax.experimental.pallas import tpu_sc as plsc

BATCH = 16384
NUM_ROWS = 1000000
EMBED_DIM = 32
NUM_CORES = 2
NUM_SUBCORES = 16
LANES = 16
NUM_WORKERS = NUM_CORES * NUM_SUBCORES
BPW = BATCH // NUM_WORKERS
CHUNK = 128
NCHUNK = BPW // CHUNK

KCOL = 1024
NBLK = NUM_ROWS // KCOL
MAIN_ROWS = NUM_ROWS - NUM_ROWS % KCOL
TAIL = NUM_ROWS - MAIN_ROWS
MAIN_N = NBLK * (KCOL // 4)
TAIL_N = TAIL // 4


def _detile_body(in_ref, out_ref):
    x = in_ref[...]
    q = x.shape[1] // 4
    parts = [jnp.transpose(x[:, i * q:(i + 1) * q]) for i in range(4)]
    out_ref[...] = jnp.concatenate(parts, axis=1)


def _detile(table_t):
    return pl.pallas_call(
        _detile_body,
        grid=(NBLK,),
        in_specs=[pl.BlockSpec((EMBED_DIM, KCOL), lambda c: (0, c))],
        out_specs=pl.BlockSpec((KCOL // 4, 128), lambda c: (c, 0)),
        out_shape=jax.ShapeDtypeStruct((MAIN_N, 128), jnp.float32),
    )(table_t)


def _detile_tail(table_t_tail):
    return pl.pallas_call(
        _detile_body,
        in_specs=[pl.BlockSpec((EMBED_DIM, TAIL), lambda: (0, 0))],
        out_specs=pl.BlockSpec((TAIL_N, 128), lambda: (0, 0)),
        out_shape=jax.ShapeDtypeStruct((TAIL_N, 128), jnp.float32),
    )(table_t_tail)


def _sc_body(uid_hbm, iid_hbm, um_hbm, im_hbm, ut_hbm, it_hbm, ub_hbm, ib_hbm,
             out_hbm, uidx_v, iidx_v, urm_v, irm_v, urt_v, irt_v, uoff_v,
             ioff_v, usel_v, isel_v, umr_v, imr_v, utr_v, itr_v,
             ub_v, ib_v, out_v, sem, bsem):
    wid = lax.axis_index("s") * NUM_CORES + lax.axis_index("c")
    base = wid * BPW

    pltpu.sync_copy(uid_hbm.at[pl.ds(base, BPW)], uidx_v)
    pltpu.sync_copy(iid_hbm.at[pl.ds(base, BPW)], iidx_v)

    cp_ub = pltpu.async_copy(ub_hbm.at[uidx_v], ub_v, bsem)
    cp_ib = pltpu.async_copy(ib_hbm.at[iidx_v], ib_v, bsem)

    def addr(j, row_v, rowt_v, off_v, sel_v):
        row_m = ((j >> 10) << 8) + (j & 255)
        row_m = jnp.minimum(row_m, MAIN_N - 1)
        off_m = ((j >> 8) & 3) << 5
        jl = jnp.maximum(j - MAIN_ROWS, 0)
        it3 = ((jl >= TAIL_N).astype(jnp.int32)
               + (jl >= 2 * TAIL_N).astype(jnp.int32)
               + (jl >= 3 * TAIL_N).astype(jnp.int32))
        rl = jl - it3 * TAIL_N
        is_tail = j >= MAIN_ROWS
        off = jnp.where(is_tail, it3 << 5, off_m)
        return row_m, rl, off, is_tail.astype(jnp.int32)

    def fill_chunk(c, idx_v, row_v, rowt_v, off_v, sel_v):
        def chunk16(t, carry):
            sl_src = pl.ds(c * CHUNK + t * LANES, LANES)
            sl_dst = pl.ds(t * LANES, LANES)
            j = idx_v[sl_src]
            rm, rl, off, sel = addr(j, row_v, rowt_v, off_v, sel_v)
            row_v[sl_dst] = rm
            rowt_v[sl_dst] = rl
            off_v[sl_dst] = off
            sel_v[sl_dst] = sel
            return carry

        lax.fori_loop(0, CHUNK // LANES, chunk16, 0)

    def process(c, carry):
        fill_chunk(c, uidx_v, urm_v, urt_v, uoff_v, usel_v)
        fill_chunk(c, iidx_v, irm_v, irt_v, ioff_v, isel_v)
        cps = [
            pltpu.async_copy(um_hbm.at[urm_v], umr_v, sem),
            pltpu.async_copy(im_hbm.at[irm_v], imr_v, sem),
            pltpu.async_copy(ut_hbm.at[urt_v], utr_v, sem),
            pltpu.async_copy(it_hbm.at[irt_v], itr_v, sem),
        ]
        for cp in cps:
            cp.wait()

        def group(t, carry2):
            sl = pl.ds(t * LANES, LANES)
            k16 = t * LANES + lax.iota(jnp.int32, LANES)
            uoff = uoff_v[sl]
            ioff = ioff_v[sl]
            umask = usel_v[sl] > 0
            imask = isel_v[sl] > 0
            acc = jnp.zeros((LANES,), jnp.float32)
            for d in range(EMBED_DIM):
                u_m = plsc.load_gather(umr_v, [k16, uoff + d])
                u_t = plsc.load_gather(utr_v, [k16, uoff + d])
                i_m = plsc.load_gather(imr_v, [k16, ioff + d])
                i_t = plsc.load_gather(itr_v, [k16, ioff + d])
                u = jnp.where(umask, u_t, u_m)
                i = jnp.where(imask, i_t, i_m)
                acc = acc + u * i
            out_v[pl.ds(c * CHUNK + t * LANES, LANES)] = acc
            return carry2

        lax.fori_loop(0, CHUNK // LANES, group, 0)
        return carry

    lax.fori_loop(0, NCHUNK, process, 0)

    cp_ub.wait()
    cp_ib.wait()

    def add_bias(t, carry):
        sl = pl.ds(t * LANES, LANES)
        out_v[sl] = out_v[sl] + ub_v[sl] + ib_v[sl]
        return carry

    lax.fori_loop(0, BPW // LANES, add_bias, 0)

    pltpu.sync_copy(out_v, out_hbm.at[pl.ds(base, BPW)])


@jax.jit
def _mf_scores(uid, iid, um, im, ut, it, ub, ib):
    mesh = plsc.VectorSubcoreMesh(core_axis_name="c", subcore_axis_name="s")
    return pl.kernel(
        _sc_body,
        out_type=jax.ShapeDtypeStruct((BATCH,), jnp.float32),
        mesh=mesh,
        compiler_params=pltpu.CompilerParams(needs_layout_passes=False),
        scratch_types=[
            pltpu.VMEM((BPW,), jnp.int32),
            pltpu.VMEM((BPW,), jnp.int32),
            pltpu.VMEM((CHUNK,), jnp.int32),
            pltpu.VMEM((CHUNK,), jnp.int32),
            pltpu.VMEM((CHUNK,), jnp.int32),
            pltpu.VMEM((CHUNK,), jnp.int32),
            pltpu.VMEM((CHUNK,), jnp.int32),
            pltpu.VMEM((CHUNK,), jnp.int32),
            pltpu.VMEM((CHUNK,), jnp.int32),
            pltpu.VMEM((CHUNK,), jnp.int32),
            pltpu.VMEM((CHUNK, 128), jnp.float32),
            pltpu.VMEM((CHUNK, 128), jnp.float32),
            pltpu.VMEM((CHUNK, 128), jnp.float32),
            pltpu.VMEM((CHUNK, 128), jnp.float32),
            pltpu.VMEM((BPW,), jnp.float32),
            pltpu.VMEM((BPW,), jnp.float32),
            pltpu.VMEM((BPW,), jnp.float32),
            pltpu.SemaphoreType.DMA,
            pltpu.SemaphoreType.DMA,
        ],
    )(uid, iid, um, im, ut, it, ub, ib)


def kernel(user_ids, item_ids, user_emb, item_emb, user_bias, item_bias):
    uid = user_ids.astype(jnp.int32)
    iid = item_ids.astype(jnp.int32)
    uet = user_emb.T
    iet = item_emb.T
    um = _detile(uet)
    im = _detile(iet)
    ut = _detile_tail(uet[:, MAIN_ROWS:])
    it = _detile_tail(iet[:, MAIN_ROWS:])
    return _mf_scores(uid, iid, um, im, ut, it,
                      user_bias.reshape(-1), item_bias.reshape(-1))


# speedup vs baseline: 5.7102x; 2.3918x over previous
import jax
import jax.numpy as jnp
from jax import lax
from jax.experimental import pallas as pl
from jax.experimental.pallas import tpu as pltpu
from jax.experimental.pallas import tpu_sc as plsc

BATCH = 16384
NUM_ROWS = 1000000
EMBED_DIM = 32
PACK = 128 // EMBED_DIM
RESHAPED_N = NUM_ROWS // PACK
NUM_CORES = 2
NUM_SUBCORES = 16
LANES = 16
NUM_WORKERS = NUM_CORES * NUM_SUBCORES
BPW = BATCH // NUM_WORKERS
CHUNK = 128
NCHUNK = BPW // CHUNK


def _sc_body(uid_hbm, iid_hbm, um_hbm, im_hbm, ub_hbm, ib_hbm,
             out_hbm, uidx_v, iidx_v,
             urow0_v, urow1_v, irow0_v, irow1_v, uoff_v, ioff_v,
             umr0_v, umr1_v, imr0_v, imr1_v,
             ub_v, ib_v, out_v, sem0, sem1, bsem):
    wid = lax.axis_index("s") * NUM_CORES + lax.axis_index("c")
    base = wid * BPW

    pltpu.sync_copy(uid_hbm.at[pl.ds(base, BPW)], uidx_v)
    pltpu.sync_copy(iid_hbm.at[pl.ds(base, BPW)], iidx_v)

    cp_ub = pltpu.async_copy(ub_hbm.at[uidx_v], ub_v, bsem)
    cp_ib = pltpu.async_copy(ib_hbm.at[iidx_v], ib_v, bsem)

    urow = (urow0_v, urow1_v)
    irow = (irow0_v, irow1_v)
    umr = (umr0_v, umr1_v)
    imr = (imr0_v, imr1_v)
    sems = (sem0, sem1)

    def fill(c, p):
        def chunk16(t, carry):
            sl_src = pl.ds(c * CHUNK + t * LANES, LANES)
            sl_dst = pl.ds(t * LANES, LANES)
            ju = uidx_v[sl_src]
            urow[p][sl_dst] = ju >> 2
            uoff_v[p, sl_dst] = (ju & 3) << 5
            ji = iidx_v[sl_src]
            irow[p][sl_dst] = ji >> 2
            ioff_v[p, sl_dst] = (ji & 3) << 5
            return carry

        lax.fori_loop(0, CHUNK // LANES, chunk16, 0)

    def issue(p):
        pltpu.async_copy(um_hbm.at[urow[p]], umr[p], sems[p])
        pltpu.async_copy(im_hbm.at[irow[p]], imr[p], sems[p])

    def wait(p):
        pltpu.make_async_copy(um_hbm.at[urow[p]], umr[p], sems[p]).wait()
        pltpu.make_async_copy(im_hbm.at[irow[p]], imr[p], sems[p]).wait()

    fill(0, 0)
    issue(0)

    for c in range(NCHUNK):
        p = c % 2
        if c + 1 < NCHUNK:
            fill(c + 1, 1 - p)
            issue(1 - p)
        wait(p)

        def group(t, carry, c=c, p=p):
            sl = pl.ds(t * LANES, LANES)
            k16 = t * LANES + lax.iota(jnp.int32, LANES)
            uoff = uoff_v[p, sl]
            ioff = ioff_v[p, sl]
            acc = jnp.zeros((LANES,), jnp.float32)
            for d in range(EMBED_DIM):
                u = plsc.load_gather(umr[p], [k16, uoff + d])
                i = plsc.load_gather(imr[p], [k16, ioff + d])
                acc = acc + u * i
            out_v[pl.ds(c * CHUNK + t * LANES, LANES)] = acc
            return carry

        lax.fori_loop(0, CHUNK // LANES, group, 0)

    cp_ub.wait()
    cp_ib.wait()

    def add_bias(t, carry):
        sl = pl.ds(t * LANES, LANES)
        out_v[sl] = out_v[sl] + ub_v[sl] + ib_v[sl]
        return carry

    lax.fori_loop(0, BPW // LANES, add_bias, 0)

    pltpu.sync_copy(out_v, out_hbm.at[pl.ds(base, BPW)])


@jax.jit
def _mf_scores(uid, iid, um, im, ub, ib):
    mesh = plsc.VectorSubcoreMesh(core_axis_name="c", subcore_axis_name="s")
    return pl.kernel(
        _sc_body,
        out_type=jax.ShapeDtypeStruct((BATCH,), jnp.float32),
        mesh=mesh,
        compiler_params=pltpu.CompilerParams(needs_layout_passes=False),
        scratch_types=[
            pltpu.VMEM((BPW,), jnp.int32),
            pltpu.VMEM((BPW,), jnp.int32),
            pltpu.VMEM((CHUNK,), jnp.int32),
            pltpu.VMEM((CHUNK,), jnp.int32),
            pltpu.VMEM((CHUNK,), jnp.int32),
            pltpu.VMEM((CHUNK,), jnp.int32),
            pltpu.VMEM((2, CHUNK), jnp.int32),
            pltpu.VMEM((2, CHUNK), jnp.int32),
            pltpu.VMEM((CHUNK, 128), jnp.float32),
            pltpu.VMEM((CHUNK, 128), jnp.float32),
            pltpu.VMEM((CHUNK, 128), jnp.float32),
            pltpu.VMEM((CHUNK, 128), jnp.float32),
            pltpu.VMEM((BPW,), jnp.float32),
            pltpu.VMEM((BPW,), jnp.float32),
            pltpu.VMEM((BPW,), jnp.float32),
            pltpu.SemaphoreType.DMA,
            pltpu.SemaphoreType.DMA,
            pltpu.SemaphoreType.DMA,
        ],
    )(uid, iid, um, im, ub, ib)


def kernel(user_ids, item_ids, user_emb, item_emb, user_bias, item_bias):
    uid = user_ids.astype(jnp.int32)
    iid = item_ids.astype(jnp.int32)
    um = user_emb.reshape(RESHAPED_N, 128)
    im = item_emb.reshape(RESHAPED_N, 128)
    return _mf_scores(uid, iid, um, im,
                      user_bias.reshape(-1), item_bias.reshape(-1))


# speedup vs baseline: 5.7102x; 1.0000x over previous
import jax
import jax.numpy as jnp
import numpy as np
from jax import lax
from jax.experimental import pallas as pl
from jax.experimental.pallas import tpu as pltpu
from jax.experimental.pallas import tpu_sc as plsc

BATCH = 16384
NUM_ROWS = 1000000
EMBED_DIM = 32
PACK = 128 // EMBED_DIM
RESHAPED_N = NUM_ROWS // PACK
NUM_CORES = 2
NUM_SUBCORES = 16
LANES = 16
NUM_WORKERS = NUM_CORES * NUM_SUBCORES
BPW = BATCH // NUM_WORKERS
CHUNK = 128
NCHUNK = BPW // CHUNK


def _sc_body(uid_hbm, iid_hbm, um_hbm, im_hbm, ub_hbm, ib_hbm,
             out_hbm, uidx_v, iidx_v,
             urow0_v, urow1_v, irow0_v, irow1_v, uoff_v, ioff_v,
             umr0_v, umr1_v, imr0_v, imr1_v,
             ub_v, ib_v, out_v, sem0, sem1, bsem):
    wid = lax.axis_index("s") * NUM_CORES + lax.axis_index("c")
    base = wid * BPW

    pltpu.sync_copy(uid_hbm.at[pl.ds(base, BPW)], uidx_v)
    pltpu.sync_copy(iid_hbm.at[pl.ds(base, BPW)], iidx_v)

    cp_ub = pltpu.async_copy(ub_hbm.at[uidx_v], ub_v, bsem)
    cp_ib = pltpu.async_copy(ib_hbm.at[iidx_v], ib_v, bsem)

    urow = (urow0_v, urow1_v)
    irow = (irow0_v, irow1_v)
    umr = (umr0_v, umr1_v)
    imr = (imr0_v, imr1_v)
    sems = (sem0, sem1)

    def fill(c, p):
        def chunk16(t, carry):
            sl_src = pl.ds(c * CHUNK + t * LANES, LANES)
            sl_dst = pl.ds(t * LANES, LANES)
            ju = uidx_v[sl_src]
            urow[p][sl_dst] = ju >> 2
            uoff_v[p, sl_dst] = (ju & 3) << 5
            ji = iidx_v[sl_src]
            irow[p][sl_dst] = ji >> 2
            ioff_v[p, sl_dst] = (ji & 3) << 5
            return carry

        lax.fori_loop(0, CHUNK // LANES, chunk16, 0)

    def issue(p):
        pltpu.async_copy(um_hbm.at[urow[p]], umr[p], sems[p])
        pltpu.async_copy(im_hbm.at[irow[p]], imr[p], sems[p])

    def wait(p):
        pltpu.make_async_copy(um_hbm.at[urow[p]], umr[p], sems[p]).wait()
        pltpu.make_async_copy(im_hbm.at[irow[p]], imr[p], sems[p]).wait()

    fill(0, 0)
    issue(0)

    for c in range(NCHUNK):
        p = c % 2
        if c + 1 < NCHUNK:
            fill(c + 1, 1 - p)
            issue(1 - p)
        wait(p)

        def group(t, carry, c=c, p=p):
            sl = pl.ds(t * LANES, LANES)
            k16 = t * LANES + lax.iota(jnp.int32, LANES)
            uoff = uoff_v[p, sl]
            ioff = ioff_v[p, sl]
            acc = jnp.zeros((LANES,), jnp.float32)
            for d in range(EMBED_DIM):
                u = plsc.load_gather(umr[p], [k16, uoff + d])
                i = plsc.load_gather(imr[p], [k16, ioff + d])
                acc = acc + u * i
            out_v[pl.ds(c * CHUNK + t * LANES, LANES)] = acc
            return carry

        lax.fori_loop(0, CHUNK // LANES, group, 0)

    cp_ub.wait()
    cp_ib.wait()

    def add_bias(t, carry):
        sl = pl.ds(t * LANES, LANES)
        out_v[sl] = out_v[sl] + ub_v[sl] + ib_v[sl]
        return carry

    lax.fori_loop(0, BPW // LANES, add_bias, 0)

    pltpu.sync_copy(out_v, out_hbm.at[pl.ds(base, BPW)])


@jax.jit
def _mf_scores(uid, iid, um, im, ub, ib):
    mesh = plsc.VectorSubcoreMesh(core_axis_name="c", subcore_axis_name="s")
    return pl.kernel(
        _sc_body,
        out_type=jax.ShapeDtypeStruct((BATCH,), jnp.float32),
        mesh=mesh,
        compiler_params=pltpu.CompilerParams(needs_layout_passes=False),
        scratch_types=[
            pltpu.VMEM((BPW,), jnp.int32),
            pltpu.VMEM((BPW,), jnp.int32),
            pltpu.VMEM((CHUNK,), jnp.int32),
            pltpu.VMEM((CHUNK,), jnp.int32),
            pltpu.VMEM((CHUNK,), jnp.int32),
            pltpu.VMEM((CHUNK,), jnp.int32),
            pltpu.VMEM((2, CHUNK), jnp.int32),
            pltpu.VMEM((2, CHUNK), jnp.int32),
            pltpu.VMEM((CHUNK, 128), jnp.float32),
            pltpu.VMEM((CHUNK, 128), jnp.float32),
            pltpu.VMEM((CHUNK, 128), jnp.float32),
            pltpu.VMEM((CHUNK, 128), jnp.float32),
            pltpu.VMEM((BPW,), jnp.float32),
            pltpu.VMEM((BPW,), jnp.float32),
            pltpu.VMEM((BPW,), jnp.float32),
            pltpu.SemaphoreType.DMA,
            pltpu.SemaphoreType.DMA,
            pltpu.SemaphoreType.DMA,
        ],
    )(uid, iid, um, im, ub, ib)


def kernel(user_ids, item_ids, user_emb, item_emb, user_bias, item_bias):
    uid = user_ids.astype(jnp.int32)
    iid = item_ids.astype(jnp.int32)
    um = user_emb.reshape(RESHAPED_N, 128) * np.float32(1.0 + 2.0 ** -24)
    im = item_emb.reshape(RESHAPED_N, 128) * np.float32(1.0 + 2.0 ** -24)
    return _mf_scores(uid, iid, um, im,
                      user_bias.reshape(-1), item_bias.reshape(-1))
